# Initial kernel scaffold; baseline (speedup 1.0000x reference)
import jax
import jax.numpy as jnp
from jax.experimental import pallas as pl


def kernel(inp, daytime, emb_day, emb_time):
    raise NotImplementedError("write your pallas kernel here")


# speedup vs baseline: 2.4088x; 2.4088x over previous
import jax
import jax.numpy as jnp
from jax import lax
from jax.experimental import pallas as pl
from jax.experimental.pallas import tpu as pltpu
from jax.experimental.pallas import tpu_sc as plsc
import functools

_B, _L, _D = 4096, 200, 64
_BL = _B * _L
_NW = 32
_TOK_PER_W = _BL // _NW
_N_TILE = 512
_TILES = _TOK_PER_W // _N_TILE
_GROUPS = _N_TILE // 16


def _make_sc_kernel():
    mesh = plsc.VectorSubcoreMesh(core_axis_name="c", subcore_axis_name="s")

    @functools.partial(
        pl.kernel,
        mesh=mesh,
        out_type=jax.ShapeDtypeStruct((_BL, 128), jnp.float32),
        scratch_types=[
            pltpu.VMEM((_N_TILE,), jnp.int32),
            pltpu.VMEM((_N_TILE,), jnp.int32),
            pltpu.VMEM((_N_TILE, 128), jnp.float32),
            pltpu.VMEM((7, 32), jnp.float32),
            pltpu.VMEM((288, 32), jnp.float32),
        ],
        compiler_params=pltpu.CompilerParams(use_tc_tiling_on_sc=False,
                                             needs_layout_passes=False),
    )
    def k(inp_hbm, didx_hbm, tidx_hbm, eday_hbm, etime_hbm, out_hbm,
          didx_v, tidx_v, out_v, eday_v, etime_v):
        wid = lax.axis_index("s") * 2 + lax.axis_index("c")
        pltpu.sync_copy(eday_hbm, eday_v)
        pltpu.sync_copy(etime_hbm, etime_v)

        def tile_body(t, carry):
            base = wid * _TOK_PER_W + t * _N_TILE
            pltpu.sync_copy(didx_hbm.at[pl.ds(base, _N_TILE)], didx_v)
            pltpu.sync_copy(tidx_hbm.at[pl.ds(base, _N_TILE)], tidx_v)
            pltpu.sync_copy(inp_hbm.at[pl.ds(base, _N_TILE), :],
                            out_v.at[:, 0:64])

            def group_body(g, carry2):
                tok = jnp.int32(16) * g + lax.iota(jnp.int32, 16)
                didx = didx_v[pl.ds(g * 16, 16)]
                tidx = tidx_v[pl.ds(g * 16, 16)]
                for j in range(32):
                    cj = jnp.full((16,), j, jnp.int32)
                    dval = plsc.load_gather(eday_v, [didx, cj])
                    plsc.store_scatter(out_v, [tok, cj + 64], dval)
                    tval = plsc.load_gather(etime_v, [tidx, cj])
                    plsc.store_scatter(out_v, [tok, cj + 96], tval)
                return carry2

            lax.fori_loop(0, _GROUPS, group_body, 0)
            pltpu.sync_copy(out_v, out_hbm.at[pl.ds(base, _N_TILE), :])
            return carry

        lax.fori_loop(0, _TILES, tile_body, 0)

    return k


_sc_kernel = _make_sc_kernel()


def kernel(inp, daytime, emb_day, emb_time):
    inp2 = inp.reshape(_BL, _D)
    didx = daytime[:, :, 0].reshape(_BL).astype(jnp.int32)
    tidx = daytime[:, :, 1].reshape(_BL).astype(jnp.int32)
    out2 = _sc_kernel(inp2, didx, tidx, emb_day, emb_time)
    return out2.reshape(_B, _L, 128)
